# baseline (speedup 1.0000x reference)
import jax
import jax.numpy as jnp
from jax import lax
from jax.experimental import pallas as pl
from jax.experimental.pallas import tpu as pltpu
from jax.experimental.pallas import tpu_sc as plsc

M = 8192
T = 8
K = 2048
NW = 32
RW = M // NW
C = 2
NCHUNK = RW // C
NPAIR = NCHUNK // 2
GC = C * T
NS = K // 16


def _body(expert_hbm, w_hbm, inv_hbm, out_hbm,
          idx_v, idx2_v, w_v, buf0, buf1, stage0, stage1,
          gsem0, gsem1):
    nc = 2
    wid = lax.axis_index("s") * nc + lax.axis_index("c")
    base_row = wid * RW
    base_g = base_row * T

    pltpu.sync_copy(inv_hbm.at[pl.ds(base_g, RW * T)], idx_v)
    pltpu.sync_copy(w_hbm.at[pl.ds(base_row, RW * T // 8)], w_v)

    def shift_body(i, _):
        idx2_v[pl.ds(i * 16, 16)] = (
            lax.shift_right_logical(idx_v[pl.ds(i * 16, 16)], 1)
        )
        return 0

    lax.fori_loop(0, RW * T // 16, shift_body, 0)

    rec_hbm = expert_hbm.bitcast(jnp.int32)

    def start_gather(c, buf, sem):
        return pltpu.async_copy(
            rec_hbm.at[idx2_v.at[pl.ds(c * GC, GC)]], buf, sem
        )

    def fold(acc):
        v = plsc.bitcast(acc, jnp.int32)
        lo = plsc.bitcast(lax.shift_left(v, 16), jnp.float32)
        hi = plsc.bitcast(v & -65536, jnp.float32)
        u = plsc.bitcast(lo + hi, jnp.int32)
        u = u + 32767 + (lax.shift_right_logical(u, 16) & 1)
        return u

    def compute(c, buf, stage):
        wmask = []
        for jl in range(GC):
            ww = w_v[c * C + jl // 8, pl.ds((jl % 8) * 16, 16)]
            par = (
                plsc.load_gather(
                    idx_v, [jnp.full((16,), c * GC + jl, jnp.int32)]
                )
                & 1
            )
            wm = jnp.where(par == 0, ww & 0xFFFF, ww & -65536)
            wmask.append(plsc.bitcast(wm, jnp.bfloat16))

        @plsc.parallel_loop(0, NS, unroll=4)
        def slice_body(s, wmask=wmask):
            k0 = s * 16
            acc0 = plsc.bitcast(buf[0, pl.ds(k0, 16)], jnp.bfloat16) * wmask[0]
            acc1 = plsc.bitcast(buf[T, pl.ds(k0, 16)], jnp.bfloat16) * wmask[T]
            for t in range(1, T):
                acc0 = acc0 + plsc.bitcast(
                    buf[t, pl.ds(k0, 16)], jnp.bfloat16
                ) * wmask[t]
                acc1 = acc1 + plsc.bitcast(
                    buf[T + t, pl.ds(k0, 16)], jnp.bfloat16
                ) * wmask[T + t]
            w0 = lax.shift_right_logical(fold(acc0), 16)
            w1 = fold(acc1) & -65536
            stage[pl.ds(k0, 16)] = w0 | w1

    outw_hbm = out_hbm.bitcast(jnp.int32)
    base_pair = base_row // 2

    start_gather(0, buf0, gsem0)

    def pair_body(cp, _):
        c0 = cp * 2
        c1 = c0 + 1
        g1 = start_gather(c1, buf1, gsem1)
        pltpu.make_async_copy(
            rec_hbm.at[idx2_v.at[pl.ds(c0 * GC, GC)]], buf0, gsem0
        ).wait()
        compute(c0, buf0, stage0)
        pltpu.sync_copy(stage0, outw_hbm.at[base_pair + c0])
        start_gather(jnp.where(c0 + 2 < NCHUNK, c0 + 2, 0), buf0, gsem0)
        g1.wait()
        compute(c1, buf1, stage1)
        pltpu.sync_copy(stage1, outw_hbm.at[base_pair + c1])
        return 0

    lax.fori_loop(0, NPAIR, pair_body, 0)
    pltpu.make_async_copy(
        rec_hbm.at[idx2_v.at[pl.ds(0, GC)]], buf0, gsem0
    ).wait()


@jax.jit
def _run(expert_bf, w_pack, inv_perm):
    mesh = plsc.VectorSubcoreMesh(core_axis_name="c", subcore_axis_name="s")
    return pl.kernel(
        _body,
        out_type=jax.ShapeDtypeStruct((M, K), jnp.bfloat16),
        mesh=mesh,
        compiler_params=pltpu.CompilerParams(
            needs_layout_passes=False, use_tc_tiling_on_sc=True
        ),
        scratch_types=[
            pltpu.VMEM((RW * T,), jnp.int32),
            pltpu.VMEM((RW * T,), jnp.int32),
            pltpu.VMEM((RW * T // 8, 128), jnp.int32),
            pltpu.VMEM((GC, K), jnp.int32),
            pltpu.VMEM((GC, K), jnp.int32),
            pltpu.VMEM((K,), jnp.int32),
            pltpu.VMEM((K,), jnp.int32),
            pltpu.SemaphoreType.DMA,
            pltpu.SemaphoreType.DMA,
        ],
    )(expert_bf, w_pack, inv_perm)


def kernel(expert_output, topk_vals, inv_perm):
    w16 = jax.lax.bitcast_convert_type(topk_vals, jnp.uint16).astype(jnp.uint32)
    w32 = ((w16 << 16) | w16).astype(jnp.int32).reshape(M * T, 1)
    w_pack = jnp.broadcast_to(w32, (M * T, 16)).reshape(M * T // 8, 128)
    return _run(expert_output, w_pack, inv_perm)


# speedup vs baseline: 1.0416x; 1.0416x over previous
import jax
import jax.numpy as jnp
from jax import lax
from jax.experimental import pallas as pl
from jax.experimental.pallas import tpu as pltpu
from jax.experimental.pallas import tpu_sc as plsc

M = 8192
T = 8
K = 2048
NW = 32
RW = M // NW
C = 2
NCHUNK = RW // C
NPAIR = NCHUNK // 2
GC = C * T
NS = K // 16


def _body(expert_hbm, w_hbm, inv_hbm, out_hbm,
          idx_v, idx2_v, w_v, buf0, buf1, stage0, stage1,
          gsem0, gsem1):
    nc = 2
    wid = lax.axis_index("s") * nc + lax.axis_index("c")
    base_row = wid * RW
    base_g = base_row * T

    pltpu.sync_copy(inv_hbm.at[pl.ds(base_g, RW * T)], idx_v)
    pltpu.sync_copy(
        w_hbm.bitcast(jnp.int32).at[
            pl.ds(pl.multiple_of(base_row // 2, 4), RW // 2)
        ],
        w_v,
    )

    def shift_body(i, _):
        idx2_v[pl.ds(i * 16, 16)] = (
            lax.shift_right_logical(idx_v[pl.ds(i * 16, 16)], 1)
        )
        return 0

    lax.fori_loop(0, RW * T // 16, shift_body, 0)

    rec_hbm = expert_hbm.bitcast(jnp.int32)

    def start_gather(c, buf, sem):
        return pltpu.async_copy(
            rec_hbm.at[idx2_v.at[pl.ds(c * GC, GC)]], buf, sem
        )

    def fold(acc):
        v = plsc.bitcast(acc, jnp.int32)
        lo = plsc.bitcast(lax.shift_left(v, 16), jnp.float32)
        hi = plsc.bitcast(v & -65536, jnp.float32)
        u = plsc.bitcast(lo + hi, jnp.int32)
        u = u + 32767 + (lax.shift_right_logical(u, 16) & 1)
        return u

    def compute(c, buf, stage):
        wmask = []
        for jl in range(GC):
            r, t = jl // T, jl % T
            ww = plsc.load_gather(
                w_v,
                [jnp.full((16,), c, jnp.int32), jnp.full((16,), t, jnp.int32)],
            )
            wlow = (
                ww & 0xFFFF if r == 0 else lax.shift_right_logical(ww, 16)
            )
            par = (
                plsc.load_gather(
                    idx_v, [jnp.full((16,), c * GC + jl, jnp.int32)]
                )
                & 1
            )
            wm = jnp.where(par == 0, wlow, lax.shift_left(wlow, 16))
            wmask.append(plsc.bitcast(wm, jnp.bfloat16))

        @plsc.parallel_loop(0, NS, unroll=2)
        def slice_body(s, wmask=wmask):
            k0 = s * 16
            acc0 = plsc.bitcast(buf[0, pl.ds(k0, 16)], jnp.bfloat16) * wmask[0]
            acc1 = plsc.bitcast(buf[T, pl.ds(k0, 16)], jnp.bfloat16) * wmask[T]
            for t in range(1, T):
                acc0 = acc0 + plsc.bitcast(
                    buf[t, pl.ds(k0, 16)], jnp.bfloat16
                ) * wmask[t]
                acc1 = acc1 + plsc.bitcast(
                    buf[T + t, pl.ds(k0, 16)], jnp.bfloat16
                ) * wmask[T + t]
            w0 = lax.shift_right_logical(fold(acc0), 16)
            w1 = fold(acc1) & -65536
            stage[pl.ds(k0, 16)] = w0 | w1

    outw_hbm = out_hbm.bitcast(jnp.int32)
    base_pair = base_row // 2

    start_gather(0, buf0, gsem0)

    def pair_body(cp, _):
        c0 = cp * 2
        c1 = c0 + 1
        g1 = start_gather(c1, buf1, gsem1)
        pltpu.make_async_copy(
            rec_hbm.at[idx2_v.at[pl.ds(c0 * GC, GC)]], buf0, gsem0
        ).wait()
        compute(c0, buf0, stage0)
        pltpu.sync_copy(stage0, outw_hbm.at[base_pair + c0])
        start_gather(jnp.where(c0 + 2 < NCHUNK, c0 + 2, 0), buf0, gsem0)
        g1.wait()
        compute(c1, buf1, stage1)
        pltpu.sync_copy(stage1, outw_hbm.at[base_pair + c1])
        return 0

    lax.fori_loop(0, NPAIR, pair_body, 0)
    pltpu.make_async_copy(
        rec_hbm.at[idx2_v.at[pl.ds(0, GC)]], buf0, gsem0
    ).wait()


@jax.jit
def _run(expert_bf, topk_vals, inv_perm):
    mesh = plsc.VectorSubcoreMesh(core_axis_name="c", subcore_axis_name="s")
    return pl.kernel(
        _body,
        out_type=jax.ShapeDtypeStruct((M, K), jnp.bfloat16),
        mesh=mesh,
        compiler_params=pltpu.CompilerParams(
            needs_layout_passes=False, use_tc_tiling_on_sc=True
        ),
        scratch_types=[
            pltpu.VMEM((RW * T,), jnp.int32),
            pltpu.VMEM((RW * T,), jnp.int32),
            pltpu.VMEM((RW // 2, T), jnp.int32),
            pltpu.VMEM((GC, K), jnp.int32),
            pltpu.VMEM((GC, K), jnp.int32),
            pltpu.VMEM((K,), jnp.int32),
            pltpu.VMEM((K,), jnp.int32),
            pltpu.SemaphoreType.DMA,
            pltpu.SemaphoreType.DMA,
        ],
    )(expert_bf, topk_vals, inv_perm)


def kernel(expert_output, topk_vals, inv_perm):
    return _run(expert_output, topk_vals, inv_perm)
